# baseline (speedup 1.0000x reference)
import jax
import jax.numpy as jnp
import numpy as np
from jax.experimental import pallas as pl
from jax.experimental.pallas import tpu as pltpu

_BLK = 1024
_NBUF = 4


def _sample(nv_ref, np_ref, rp_ref, uc_ref):
    k_sizes = np_ref.shape[1]
    total = np_ref[0, 0]
    for k in range(1, k_sizes):
        total = total + np_ref[0, k]
    u = uc_ref[0]
    idx = 0
    csums = []
    acc = None
    for k in range(k_sizes):
        p = np_ref[0, k] / total
        acc = p if acc is None else acc + p
        csums.append(acc)
    for k in range(k_sizes - 1, -1, -1):
        idx = jnp.where(csums[k] >= u, k, idx)
    return nv_ref[0, idx], rp_ref[0, idx]


def _make_body(num_steps, blk):
    def _body(bi_ref, bj_ref, nv_ref, np_ref, rp_ref, uc_ref, u_hbm, out_ref,
              ubuf, obuf, tbuf, zbuf, isems, osems):
        k = pl.program_id(0)
        slot = jax.lax.rem(k, _NBUF)
        islot = jax.lax.rem(k, 2)
        bi = bi_ref[k]
        bj = bj_ref[k]

        n_nodes, r = _sample(nv_ref, np_ref, rp_ref, uc_ref)
        nf = n_nodes

        def needed(step):
            pb = bi_ref[step]
            qb = bj_ref[step]
            return ((pb * blk).astype(jnp.float32) < nf) & (
                (qb * blk).astype(jnp.float32) < nf)

        def start_in(step, s):
            pltpu.make_async_copy(
                u_hbm.at[pl.ds(bi_ref[step] * blk, blk),
                         pl.ds(bj_ref[step] * blk, blk)],
                ubuf.at[s],
                isems.at[s],
            ).start()

        @pl.when(k == 0)
        def _():
            zbuf[...] = jnp.zeros_like(zbuf)

            @pl.when(needed(0))
            def _():
                start_in(0, islot)

        @pl.when(k + 1 < num_steps)
        def _():
            @pl.when(needed(k + 1))
            def _():
                start_in(k + 1, jax.lax.rem(k + 1, 2))

        def _wait_step(step, wslot):
            pb = bi_ref[step]
            qb = bj_ref[step]
            pltpu.make_async_copy(
                obuf.at[wslot],
                out_ref.at[pl.ds(pb * blk, blk), pl.ds(qb * blk, blk)],
                osems.at[wslot, 0],
            ).wait()

            @pl.when(pb != qb)
            def _():
                pltpu.make_async_copy(
                    tbuf.at[wslot],
                    out_ref.at[pl.ds(qb * blk, blk), pl.ds(pb * blk, blk)],
                    osems.at[wslot, 1],
                ).wait()

        @pl.when(k >= _NBUF)
        def _():
            _wait_step(k - _NBUF, slot)

        blk_needed = needed(k)

        @pl.when(blk_needed)
        def _():
            pltpu.make_async_copy(
                u_hbm.at[pl.ds(bi * blk, blk), pl.ds(bj * blk, blk)],
                ubuf.at[islot],
                isems.at[islot],
            ).wait()

            row = (bi * blk
                   + jax.lax.broadcasted_iota(jnp.int32, (blk, blk), 0)
                   ).astype(jnp.float32)
            col = (bj * blk
                   + jax.lax.broadcasted_iota(jnp.int32, (blk, blk), 1)
                   ).astype(jnp.float32)
            u = ubuf[islot]
            lu = ((u <= r) & (col > row) & (row < nf) & (col < nf)
                  ).astype(jnp.int32)
            lt = lu.T

            @pl.when(bi == bj)
            def _():
                obuf[slot] = lu | lt

            @pl.when(bi != bj)
            def _():
                obuf[slot] = lu
                tbuf[slot] = lt

            pltpu.make_async_copy(
                obuf.at[slot],
                out_ref.at[pl.ds(bi * blk, blk), pl.ds(bj * blk, blk)],
                osems.at[slot, 0],
            ).start()

            @pl.when(bi != bj)
            def _():
                pltpu.make_async_copy(
                    tbuf.at[slot],
                    out_ref.at[pl.ds(bj * blk, blk), pl.ds(bi * blk, blk)],
                    osems.at[slot, 1],
                ).start()

        @pl.when(jnp.logical_not(blk_needed))
        def _():
            pltpu.make_async_copy(
                zbuf,
                out_ref.at[pl.ds(bi * blk, blk), pl.ds(bj * blk, blk)],
                osems.at[slot, 0],
            ).start()

            @pl.when(bi != bj)
            def _():
                pltpu.make_async_copy(
                    zbuf,
                    out_ref.at[pl.ds(bj * blk, blk), pl.ds(bi * blk, blk)],
                    osems.at[slot, 1],
                ).start()

        @pl.when(k == num_steps - 1)
        def _():
            for s in range(max(0, num_steps - _NBUF), num_steps):
                _wait_step(s, s % _NBUF)

    return _body


def kernel(N_values, N_probs, r_probs, u_cat, u_links):
    n = u_links.shape[0]
    g = n // _BLK
    pairs = [(i, j) for i in range(g) for j in range(i, g)]
    num_steps = len(pairs)
    bi_arr = jnp.asarray(np.array([p[0] for p in pairs], dtype=np.int32))
    bj_arr = jnp.asarray(np.array([p[1] for p in pairs], dtype=np.int32))

    grid_spec = pltpu.PrefetchScalarGridSpec(
        num_scalar_prefetch=6,
        grid=(num_steps,),
        in_specs=[
            pl.BlockSpec(memory_space=pl.ANY),
        ],
        out_specs=pl.BlockSpec(memory_space=pl.ANY),
        scratch_shapes=[
            pltpu.VMEM((2, _BLK, _BLK), jnp.float32),
            pltpu.VMEM((_NBUF, _BLK, _BLK), jnp.int32),
            pltpu.VMEM((_NBUF, _BLK, _BLK), jnp.int32),
            pltpu.VMEM((_BLK, _BLK), jnp.int32),
            pltpu.SemaphoreType.DMA((2,)),
            pltpu.SemaphoreType.DMA((_NBUF, 2)),
        ],
    )

    return pl.pallas_call(
        _make_body(num_steps, _BLK),
        grid_spec=grid_spec,
        out_shape=jax.ShapeDtypeStruct((n, n), jnp.int32),
    )(
        bi_arr,
        bj_arr,
        N_values.reshape(1, 16),
        N_probs.reshape(1, 16),
        r_probs.reshape(1, 16),
        u_cat.reshape(1),
        u_links,
    )


# speedup vs baseline: 1.0040x; 1.0040x over previous
import jax
import jax.numpy as jnp
import numpy as np
from jax.experimental import pallas as pl
from jax.experimental.pallas import tpu as pltpu

_BLK = 1024
_NBUF = 3


def _sample(nv_ref, np_ref, rp_ref, uc_ref):
    k_sizes = np_ref.shape[1]
    total = np_ref[0, 0]
    for k in range(1, k_sizes):
        total = total + np_ref[0, k]
    u = uc_ref[0]
    idx = 0
    csums = []
    acc = None
    for k in range(k_sizes):
        p = np_ref[0, k] / total
        acc = p if acc is None else acc + p
        csums.append(acc)
    for k in range(k_sizes - 1, -1, -1):
        idx = jnp.where(csums[k] >= u, k, idx)
    return nv_ref[0, idx], rp_ref[0, idx]


def _make_body(num_steps, blk):
    def _body(bi_ref, bj_ref, nv_ref, np_ref, rp_ref, uc_ref, u_hbm, out_ref,
              ubuf, obuf, tbuf, zbuf, isems, osems):
        k = pl.program_id(0)
        slot = jax.lax.rem(k, _NBUF)
        islot = jax.lax.rem(k, 2)
        bi = bi_ref[k]
        bj = bj_ref[k]

        n_nodes, r = _sample(nv_ref, np_ref, rp_ref, uc_ref)
        nf = n_nodes

        def needed(step):
            pb = bi_ref[step]
            qb = bj_ref[step]
            return ((pb * blk).astype(jnp.float32) < nf) & (
                (qb * blk).astype(jnp.float32) < nf)

        def start_in(step, s):
            pltpu.make_async_copy(
                u_hbm.at[pl.ds(bi_ref[step] * blk, blk),
                         pl.ds(bj_ref[step] * blk, blk)],
                ubuf.at[s],
                isems.at[s],
            ).start()

        @pl.when(k == 0)
        def _():
            zbuf[...] = jnp.zeros_like(zbuf)

            @pl.when(needed(0))
            def _():
                start_in(0, islot)

        @pl.when(k + 1 < num_steps)
        def _():
            @pl.when(needed(k + 1))
            def _():
                start_in(k + 1, jax.lax.rem(k + 1, 2))

        def _wait_step(step, wslot):
            pb = bi_ref[step]
            qb = bj_ref[step]
            pltpu.make_async_copy(
                obuf.at[wslot],
                out_ref.at[pl.ds(pb * blk, blk), pl.ds(qb * blk, blk)],
                osems.at[wslot, 0],
            ).wait()

            @pl.when(pb != qb)
            def _():
                pltpu.make_async_copy(
                    tbuf.at[wslot],
                    out_ref.at[pl.ds(qb * blk, blk), pl.ds(pb * blk, blk)],
                    osems.at[wslot, 1],
                ).wait()

        @pl.when(k >= _NBUF)
        def _():
            _wait_step(k - _NBUF, slot)

        blk_needed = needed(k)

        @pl.when(blk_needed)
        def _():
            pltpu.make_async_copy(
                u_hbm.at[pl.ds(bi * blk, blk), pl.ds(bj * blk, blk)],
                ubuf.at[islot],
                isems.at[islot],
            ).wait()

            row = (bi * blk
                   + jax.lax.broadcasted_iota(jnp.int32, (blk, blk), 0)
                   ).astype(jnp.float32)
            col = (bj * blk
                   + jax.lax.broadcasted_iota(jnp.int32, (blk, blk), 1)
                   ).astype(jnp.float32)
            u = ubuf[islot]
            lu = ((u <= r) & (col > row) & (row < nf) & (col < nf)
                  ).astype(jnp.int32)
            lt = lu.T

            @pl.when(bi == bj)
            def _():
                obuf[slot] = lu | lt

            @pl.when(bi != bj)
            def _():
                obuf[slot] = lu
                tbuf[slot] = lt

            pltpu.make_async_copy(
                obuf.at[slot],
                out_ref.at[pl.ds(bi * blk, blk), pl.ds(bj * blk, blk)],
                osems.at[slot, 0],
            ).start()

            @pl.when(bi != bj)
            def _():
                pltpu.make_async_copy(
                    tbuf.at[slot],
                    out_ref.at[pl.ds(bj * blk, blk), pl.ds(bi * blk, blk)],
                    osems.at[slot, 1],
                ).start()

        @pl.when(jnp.logical_not(blk_needed))
        def _():
            pltpu.make_async_copy(
                zbuf,
                out_ref.at[pl.ds(bi * blk, blk), pl.ds(bj * blk, blk)],
                osems.at[slot, 0],
            ).start()

            @pl.when(bi != bj)
            def _():
                pltpu.make_async_copy(
                    zbuf,
                    out_ref.at[pl.ds(bj * blk, blk), pl.ds(bi * blk, blk)],
                    osems.at[slot, 1],
                ).start()

        @pl.when(k == num_steps - 1)
        def _():
            for s in range(max(0, num_steps - _NBUF), num_steps):
                _wait_step(s, s % _NBUF)

    return _body


def kernel(N_values, N_probs, r_probs, u_cat, u_links):
    n = u_links.shape[0]
    g = n // _BLK
    pairs = [(i, j) for i in range(g) for j in range(i, g)]
    num_steps = len(pairs)
    bi_arr = jnp.asarray(np.array([p[0] for p in pairs], dtype=np.int32))
    bj_arr = jnp.asarray(np.array([p[1] for p in pairs], dtype=np.int32))

    grid_spec = pltpu.PrefetchScalarGridSpec(
        num_scalar_prefetch=6,
        grid=(num_steps,),
        in_specs=[
            pl.BlockSpec(memory_space=pl.ANY),
        ],
        out_specs=pl.BlockSpec(memory_space=pl.ANY),
        scratch_shapes=[
            pltpu.VMEM((2, _BLK, _BLK), jnp.float32),
            pltpu.VMEM((_NBUF, _BLK, _BLK), jnp.int32),
            pltpu.VMEM((_NBUF, _BLK, _BLK), jnp.int32),
            pltpu.VMEM((_BLK, _BLK), jnp.int32),
            pltpu.SemaphoreType.DMA((2,)),
            pltpu.SemaphoreType.DMA((_NBUF, 2)),
        ],
    )

    return pl.pallas_call(
        _make_body(num_steps, _BLK),
        grid_spec=grid_spec,
        out_shape=jax.ShapeDtypeStruct((n, n), jnp.int32),
    )(
        bi_arr,
        bj_arr,
        N_values.reshape(1, 16),
        N_probs.reshape(1, 16),
        r_probs.reshape(1, 16),
        u_cat.reshape(1),
        u_links,
    )
